# baseline (speedup 1.0000x reference)
import functools

import jax
import jax.numpy as jnp
from jax import lax
from jax.experimental import pallas as pl
from jax.experimental.pallas import tpu as pltpu
from jax.experimental.pallas import tpu_sc as plsc

_NUM_CORES = 2
_NUM_SUBCORES = 16
_NW = _NUM_CORES * _NUM_SUBCORES
_C = 256
_G = 16
_PAD = 128


def _sc_gather_t(uidx, iidx, ug_t, um_t, ig_t, im_t):
    b = uidx.shape[0]
    d = ug_t.shape[0]
    n = ug_t.shape[1]
    n_pad = -(-n // 128) * 128
    per_w = -(-n // _NW)
    nch = -(-per_w // _C) + 1
    nch += nch % 2
    cpg = -(-nch // _G)
    gshift = (cpg * _C).bit_length() - 1
    assert cpg * _C == 1 << gshift
    hmax = b + 16

    mesh = plsc.VectorSubcoreMesh(
        core_axis_name="c", subcore_axis_name="s")
    out_type = [jax.ShapeDtypeStruct((b + _PAD, 2 * d), jnp.float32)] * 4

    @functools.partial(
        pl.kernel,
        out_type=out_type,
        mesh=mesh,
        compiler_params=pltpu.CompilerParams(
            needs_layout_passes=False, disable_bounds_checks=True),
        scratch_types=[
            pltpu.SMEM((_G + 1,), jnp.int32),
            pltpu.VMEM((2048,), jnp.int32),
            pltpu.VMEM((hmax,), jnp.int32),
            pltpu.VMEM((hmax,), jnp.int32),
            pltpu.VMEM((d, _C), jnp.float32),
            pltpu.VMEM((d, _C), jnp.float32),
            pltpu.VMEM((d, _C), jnp.float32),
            pltpu.VMEM((d, _C), jnp.float32),
            pltpu.VMEM((16, 2 * d), jnp.float32),
            pltpu.VMEM((16, 2 * d), jnp.float32),
            pltpu.VMEM((16, 2 * d), jnp.float32),
            pltpu.VMEM((16, 2 * d), jnp.float32),
            pltpu.VMEM((16, 2 * d), jnp.float32),
            pltpu.VMEM((16, 2 * d), jnp.float32),
            pltpu.VMEM((16, 2 * d), jnp.float32),
            pltpu.VMEM((16, 2 * d), jnp.float32),
            pltpu.SemaphoreType.DMA,
            pltpu.SemaphoreType.DMA,
            pltpu.SemaphoreType.DMA,
            pltpu.SemaphoreType.DMA,
            pltpu.SemaphoreType.DMA,
            pltpu.SemaphoreType.DMA,
        ],
    )
    def gather_kernel(uidx_hbm, iidx_hbm, ug_hbm, um_hbm, ig_hbm, im_hbm,
                      ug_out, um_out, ig_out, im_out,
                      goff_s, idsb, hit_p, bkt_p,
                      cb00, cb01, cb10, cb11,
                      ro00, ro01, ro10, ro11, ro20, ro21, ro30, ro31,
                      sem_c0, sem_c1, sem_s0, sem_s1, sem_s2, sem_s3):
        wid = lax.axis_index("s") * _NUM_CORES + lax.axis_index("c")
        lo = (wid * per_w) // 128 * 128
        csems = (sem_c0, sem_c1)
        ssems = (sem_s0, sem_s1, sem_s2, sem_s3)
        cbufs = ((cb00, cb01), (cb10, cb11))
        rbufs = ((ro00, ro01), (ro10, ro11), (ro20, ro21), (ro30, ro31))
        act_p = hit_p
        iota = lax.iota(jnp.int32, 16)

        def scan_hits(idx_hbm):
            lo_v = jnp.broadcast_to(lo, (16,))
            hi_v = jnp.broadcast_to(lo + nch * _C, (16,))
            off = jnp.int32(0)
            for hb in range(b // 2048):
                pltpu.sync_copy(
                    idx_hbm.at[pl.ds(hb * 2048, 2048)], idsb)

                def blk(t, off, hb=hb):
                    v = idsb[pl.ds(t * 16, 16)]
                    m = (v >= lo_v) & (v < hi_v)
                    k = plsc.all_reduce_population_count(m)[0]

                    @pl.when(k > 0)
                    def _():
                        p = ((v - lo_v) << 14) | (
                            iota + (hb * 2048 + t * 16))
                        plsc.store_compressed(
                            hit_p.at[pl.ds(off, 16)], p, mask=m)
                    return off + k

                off = lax.fori_loop(0, 128, blk, off)
            return off

        def bucket_hits(nh):
            nhb = (nh + 15) // 16

            def one_group(g, off):
                goff_s[g] = off
                gv = jnp.broadcast_to(g, (16,))

                def blk(t, o):
                    p = hit_p[pl.ds(t * 16, 16)]
                    m = (p >> (14 + gshift)) == gv
                    k = plsc.all_reduce_population_count(m)[0]

                    @pl.when(k > 0)
                    def _():
                        plsc.store_compressed(
                            bkt_p.at[pl.ds(o, 16)], p, mask=m)
                    return o + k

                return lax.fori_loop(0, nhb, blk, off)

            hit_p[pl.ds(nh, 16)] = jnp.broadcast_to(jnp.int32(-1), (16,))
            off = jnp.int32(0)
            for g in range(_G):
                off = one_group(g, off)
            goff_s[_G] = off
            bkt_p[pl.ds(off, 16)] = jnp.broadcast_to(jnp.int32(-1), (16,))

        def c_start(c):
            return jnp.minimum(lo + c * _C, n_pad - _C)

        def fire(c, phase, tbls):
            c0 = pl.multiple_of(c_start(c), 128)
            for u, tbl in enumerate(tbls):
                pltpu.async_copy(
                    tbl.at[:, pl.ds(c0, _C)], cbufs[phase][u],
                    csems[phase])

        def drain_chunk(phase, tbls):
            for u, tbl in enumerate(tbls):
                pltpu.make_async_copy(
                    tbl.at[:, pl.ds(0, _C)], cbufs[phase][u],
                    csems[phase]).wait()

        def drain_scat(slot, outs, cnt):
            def one(_, carry):
                for u in range(2):
                    pltpu.make_async_copy(
                        outs[u].at[pl.ds(0, 16)], rbufs[slot][u],
                        ssems[slot]).wait()
                return carry
            lax.fori_loop(0, cnt, one, jnp.int32(0))

        def process(c, phase, slot, outs):
            c0 = c_start(c)
            lo_c = jnp.broadcast_to(lo + c * _C, (16,))
            hi_c = jnp.broadcast_to(
                jnp.minimum(lo + (c + 1) * _C, n), (16,))
            g = c // cpg
            b0 = goff_s[g]
            sz = goff_s[g + 1] - b0
            lo_b = jnp.broadcast_to(lo, (16,))

            def blk(t, o):
                p = bkt_p[pl.ds(b0 + t * 16, 16)]
                v = (p >> 14) + lo_b
                m = (v >= lo_c) & (v < hi_c)
                k = plsc.all_reduce_population_count(m)[0]

                @pl.when(k > 0)
                def _():
                    plsc.store_compressed(
                        act_p.at[pl.ds(o, 16)], p, mask=m)
                return o + k

            ka = lax.fori_loop(0, (sz + 15) // 16, blk, jnp.int32(0))

            def grp(g2, cnt):
                @pl.when(g2 > 0)
                def _():
                    for u in range(2):
                        pltpu.make_async_copy(
                            outs[u].at[pl.ds(0, 16)], rbufs[slot][u],
                            ssems[slot]).wait()
                p = act_p[pl.ds(g2 * 16, 16)]
                v = (p >> 14) + lo_b
                jv = p & jnp.broadcast_to(jnp.int32(0x3FFF), (16,))
                for q in range(16):
                    col = jnp.clip(v[q] - c0, 0, _C - 1)
                    colv = jnp.broadcast_to(col, (16,)).astype(jnp.int32)
                    for u in range(2):
                        for ss in range(d // 16):
                            rr = iota + 16 * ss
                            val = plsc.load_gather(
                                cbufs[phase][u], [rr, colv])
                            rbufs[slot][u][q, pl.ds(16 * ss, 16)] = val
                rem = jnp.broadcast_to(ka - g2 * 16, (16,))
                jfin = jnp.where(iota < rem, jv, b + iota)
                for u in range(2):
                    pltpu.async_copy(
                        rbufs[slot][u], outs[u].at[jfin], ssems[slot])
                return jnp.int32(1)

            return lax.fori_loop(0, (ka + 15) // 16, grp, jnp.int32(0))

        def sweep(idx_hbm, tbls, outs):
            fire(0, 0, tbls)
            fire(1, 1, tbls)
            nh = scan_hits(idx_hbm)
            bucket_hits(nh)

            def body(i, cnts):
                cl = list(cnts)
                for sub in range(4):
                    phase = sub % 2
                    c = 4 * i + sub
                    drain_chunk(phase, tbls)
                    drain_scat(sub, outs, cl[sub])
                    cl[sub] = process(c, phase, sub, outs)
                    fire(jnp.minimum(c + 2, nch - 1), phase, tbls)
                return tuple(cl)

            z = jnp.int32(0)
            cnts = lax.fori_loop(0, nch // 4, body, (z, z, z, z))
            drain_chunk(0, tbls)
            drain_chunk(1, tbls)
            for sub in range(4):
                drain_scat(sub, outs, cnts[sub])

        sweep(uidx_hbm, (ug_hbm, um_hbm), (ug_out, um_out))
        sweep(iidx_hbm, (ig_hbm, im_hbm), (ig_out, im_out))

    return gather_kernel(uidx, iidx, ug_t, um_t, ig_t, im_t)


def _mlp_body(ug, ig, um, im, w1u, w1i, b1, w2t, b2, wlg, wlm, bl, out):
    d = w2t.shape[0]
    gmf = ug[:, :d] * ig[:, :d]
    h = (jnp.dot(um[:, :d], w1u[:], preferred_element_type=jnp.float32)
         + jnp.dot(im[:, :d], w1i[:], preferred_element_type=jnp.float32)
         + b1[:])
    h = jnp.maximum(h, 0.0)
    mlp = jnp.dot(h, w2t[:], preferred_element_type=jnp.float32) + b2[:]
    z = (jnp.dot(gmf, wlg[:], preferred_element_type=jnp.float32)
         + jnp.dot(mlp, wlm[:], preferred_element_type=jnp.float32)
         + bl[:])
    out[:] = jax.nn.sigmoid(z)


def kernel(user_ids, item_ids, U_gmf, I_gmf, U_mlp, I_mlp,
           W1, b1, W2, b2, Wl, bl):
    b = user_ids.shape[0]
    d = U_gmf.shape[1]
    uids = user_ids.astype(jnp.int32)
    iids = item_ids.astype(jnp.int32)

    ug, um, ig, im = _sc_gather_t(
        uids, iids, U_gmf.T, U_mlp.T, I_gmf.T, I_mlp.T)

    w1u = W1[:, :d].T
    w1i = W1[:, d:].T
    w2t = W2.T
    wlg = Wl[0, :d].reshape(d, 1)
    wlm = Wl[0, d:].reshape(d, 1)
    b1r = b1.reshape(1, d)
    b2r = b2.reshape(1, d)
    blr = bl.reshape(1, 1)

    bb = 2048
    grid = (b // bb,)
    row_spec = pl.BlockSpec((bb, 2 * d), lambda i: (i, 0))
    full = lambda shape: pl.BlockSpec(shape, lambda i: (0, 0))

    return pl.pallas_call(
        _mlp_body,
        grid=grid,
        in_specs=[
            row_spec, row_spec, row_spec, row_spec,
            full((d, d)), full((d, d)), full((1, d)),
            full((d, d)), full((1, d)),
            full((d, 1)), full((d, 1)), full((1, 1)),
        ],
        out_specs=pl.BlockSpec((bb, 1), lambda i: (i, 0)),
        out_shape=jax.ShapeDtypeStruct((b, 1), jnp.float32),
    )(ug, ig, um, im, w1u, w1i, b1r, w2t, b2r, wlg, wlm, blr)


# speedup vs baseline: 1.0853x; 1.0853x over previous
import functools

import jax
import jax.numpy as jnp
from jax import lax
from jax.experimental import pallas as pl
from jax.experimental.pallas import tpu as pltpu
from jax.experimental.pallas import tpu_sc as plsc

_NUM_CORES = 2
_NUM_SUBCORES = 16
_NW = _NUM_CORES * _NUM_SUBCORES
_C = 256
_PAD = 128


def _sc_gather_t(uidx, iidx, ug_t, um_t, ig_t, im_t):
    b = uidx.shape[0]
    d = ug_t.shape[0]
    n = ug_t.shape[1]
    n_pad = -(-n // 128) * 128
    per_w = -(-n // _NW)
    nch = -(-per_w // _C) + 1
    nch += nch % 2
    hmax = b + 16

    mesh = plsc.VectorSubcoreMesh(
        core_axis_name="c", subcore_axis_name="s")
    out_type = [jax.ShapeDtypeStruct((b + _PAD, 2 * d), jnp.float32)] * 4

    @functools.partial(
        pl.kernel,
        out_type=out_type,
        mesh=mesh,
        compiler_params=pltpu.CompilerParams(
            needs_layout_passes=False, disable_bounds_checks=True),
        scratch_types=[
            pltpu.VMEM((b,), jnp.int32),
            pltpu.VMEM((hmax,), jnp.int32),
            pltpu.VMEM((hmax,), jnp.int32),
            pltpu.VMEM((d, _C), jnp.float32),
            pltpu.VMEM((d, _C), jnp.float32),
            pltpu.VMEM((d, _C), jnp.float32),
            pltpu.VMEM((d, _C), jnp.float32),
            pltpu.VMEM((16, 2 * d), jnp.float32),
            pltpu.VMEM((16, 2 * d), jnp.float32),
            pltpu.VMEM((16, 2 * d), jnp.float32),
            pltpu.VMEM((16, 2 * d), jnp.float32),
            pltpu.SemaphoreType.DMA,
            pltpu.SemaphoreType.DMA,
            pltpu.SemaphoreType.DMA,
            pltpu.SemaphoreType.DMA,
        ],
    )
    def gather_kernel(uidx_hbm, iidx_hbm, ug_hbm, um_hbm, ig_hbm, im_hbm,
                      ug_out, um_out, ig_out, im_out,
                      ids_v, hit_j, act_j,
                      cb00, cb01, cb10, cb11, ro00, ro01, ro10, ro11,
                      sem_c0, sem_c1, sem_s0, sem_s1):
        wid = lax.axis_index("s") * _NUM_CORES + lax.axis_index("c")
        lo = (wid * per_w) // 128 * 128
        csems = (sem_c0, sem_c1)
        ssems = (sem_s0, sem_s1)
        cbufs = ((cb00, cb01), (cb10, cb11))
        rbufs = ((ro00, ro01), (ro10, ro11))
        iota = lax.iota(jnp.int32, 16)

        def scan_hits(idx_hbm):
            pltpu.sync_copy(idx_hbm, ids_v)
            lo_v = jnp.broadcast_to(lo, (16,))
            hi_v = jnp.broadcast_to(lo + nch * _C, (16,))

            def blk(t, off):
                v = ids_v[pl.ds(t * 16, 16)]
                m = (v >= lo_v) & (v < hi_v)
                k = plsc.all_reduce_population_count(m)[0]

                @pl.when(k > 0)
                def _():
                    plsc.store_compressed(
                        hit_j.at[pl.ds(off, 16)], iota + t * 16, mask=m)
                return off + k

            nh = lax.fori_loop(0, b // 16, blk, jnp.int32(0))
            hit_j[pl.ds(nh, 16)] = jnp.broadcast_to(jnp.int32(0), (16,))
            return nh

        def c_start(c):
            return jnp.minimum(lo + c * _C, n_pad - _C)

        def fire(c, phase, tbls):
            c0 = pl.multiple_of(c_start(c), 128)
            for u, tbl in enumerate(tbls):
                pltpu.async_copy(
                    tbl.at[:, pl.ds(c0, _C)], cbufs[phase][u],
                    csems[phase])

        def drain_chunk(phase, tbls):
            for u, tbl in enumerate(tbls):
                pltpu.make_async_copy(
                    tbl.at[:, pl.ds(0, _C)], cbufs[phase][u],
                    csems[phase]).wait()

        def drain_scats(phase, outs, cnt):
            def one(_, carry):
                pltpu.make_async_copy(
                    outs[0].at[pl.ds(0, 16)], rbufs[phase][0],
                    ssems[phase]).wait()
                return carry
            lax.fori_loop(0, cnt, one, jnp.int32(0))

        def process(c, phase, nh, outs):
            c0 = c_start(c)
            lo_c = jnp.broadcast_to(lo + c * _C, (16,))
            hi_c = jnp.broadcast_to(
                jnp.minimum(lo + (c + 1) * _C, n), (16,))

            def blk(t, off):
                jv = hit_j[pl.ds(t * 16, 16)]
                v = plsc.load_gather(ids_v, [jv])
                m = (v >= lo_c) & (v < hi_c)
                k = plsc.all_reduce_population_count(m)[0]

                @pl.when(k > 0)
                def _():
                    plsc.store_compressed(
                        act_j.at[pl.ds(off, 16)], jv, mask=m)
                return off + k

            nblk = (nh + 15) // 16
            ka = lax.fori_loop(0, nblk, blk, jnp.int32(0))
            act_j[pl.ds(ka, 16)] = jnp.broadcast_to(jnp.int32(0), (16,))

            def grp(g, cnt):
                @pl.when(g > 0)
                def _():
                    for u in range(2):
                        pltpu.make_async_copy(
                            outs[u].at[pl.ds(0, 16)], rbufs[phase][u],
                            ssems[phase]).wait()
                aj = act_j[pl.ds(g * 16, 16)]
                av = plsc.load_gather(ids_v, [aj])
                for q in range(16):
                    col = jnp.clip(av[q] - c0, 0, _C - 1)
                    colv = jnp.broadcast_to(col, (16,)).astype(jnp.int32)
                    for u in range(2):
                        for s in range(d // 16):
                            rr = iota + 16 * s
                            val = plsc.load_gather(
                                cbufs[phase][u], [rr, colv])
                            rbufs[phase][u][q, pl.ds(16 * s, 16)] = val
                rem = jnp.broadcast_to(ka - g * 16, (16,))
                jfin = jnp.where(iota < rem, aj, b + iota)
                for u in range(2):
                    pltpu.async_copy(
                        rbufs[phase][u], outs[u].at[jfin], ssems[phase])
                return jnp.int32(2)

            ngrp = (ka + 15) // 16
            return lax.fori_loop(0, ngrp, grp, jnp.int32(0))

        def sweep(idx_hbm, tbls, outs):
            nh = scan_hits(idx_hbm)
            fire(0, 0, tbls)
            fire(1, 1, tbls)

            def body(i, cnts):
                cnt0, cnt1 = cnts
                drain_chunk(0, tbls)
                drain_scats(0, outs, cnt0)
                cnt0 = process(2 * i, 0, nh, outs)
                fire(jnp.minimum(2 * i + 2, nch - 1), 0, tbls)
                drain_chunk(1, tbls)
                drain_scats(1, outs, cnt1)
                cnt1 = process(2 * i + 1, 1, nh, outs)
                fire(jnp.minimum(2 * i + 3, nch - 1), 1, tbls)
                return (cnt0, cnt1)

            cnt0, cnt1 = lax.fori_loop(
                0, nch // 2, body, (jnp.int32(0), jnp.int32(0)))
            drain_chunk(0, tbls)
            drain_scats(0, outs, cnt0)
            drain_chunk(1, tbls)
            drain_scats(1, outs, cnt1)

        sweep(uidx_hbm, (ug_hbm, um_hbm), (ug_out, um_out))
        sweep(iidx_hbm, (ig_hbm, im_hbm), (ig_out, im_out))

    return gather_kernel(uidx, iidx, ug_t, um_t, ig_t, im_t)


def _mlp_body(ug, ig, um, im, w1u, w1i, b1, w2t, b2, wlg, wlm, bl, out):
    d = w2t.shape[0]
    gmf = ug[:, :d] * ig[:, :d]
    h = (jnp.dot(um[:, :d], w1u[:], preferred_element_type=jnp.float32)
         + jnp.dot(im[:, :d], w1i[:], preferred_element_type=jnp.float32)
         + b1[:])
    h = jnp.maximum(h, 0.0)
    mlp = jnp.dot(h, w2t[:], preferred_element_type=jnp.float32) + b2[:]
    z = (jnp.dot(gmf, wlg[:], preferred_element_type=jnp.float32)
         + jnp.dot(mlp, wlm[:], preferred_element_type=jnp.float32)
         + bl[:])
    out[:] = jax.nn.sigmoid(z)


def kernel(user_ids, item_ids, U_gmf, I_gmf, U_mlp, I_mlp,
           W1, b1, W2, b2, Wl, bl):
    b = user_ids.shape[0]
    d = U_gmf.shape[1]
    uids = user_ids.astype(jnp.int32)
    iids = item_ids.astype(jnp.int32)

    ug, um, ig, im = _sc_gather_t(
        uids, iids, U_gmf.T, U_mlp.T, I_gmf.T, I_mlp.T)

    w1u = W1[:, :d].T
    w1i = W1[:, d:].T
    w2t = W2.T
    wlg = Wl[0, :d].reshape(d, 1)
    wlm = Wl[0, d:].reshape(d, 1)
    b1r = b1.reshape(1, d)
    b2r = b2.reshape(1, d)
    blr = bl.reshape(1, 1)

    bb = 2048
    grid = (b // bb,)
    row_spec = pl.BlockSpec((bb, 2 * d), lambda i: (i, 0))
    full = lambda shape: pl.BlockSpec(shape, lambda i: (0, 0))

    return pl.pallas_call(
        _mlp_body,
        grid=grid,
        in_specs=[
            row_spec, row_spec, row_spec, row_spec,
            full((d, d)), full((d, d)), full((1, d)),
            full((d, d)), full((1, d)),
            full((d, 1)), full((d, 1)), full((1, 1)),
        ],
        out_specs=pl.BlockSpec((bb, 1), lambda i: (i, 0)),
        out_shape=jax.ShapeDtypeStruct((b, 1), jnp.float32),
    )(ug, ig, um, im, w1u, w1i, b1r, w2t, b2r, wlg, wlm, blr)
